# Initial kernel scaffold; baseline (speedup 1.0000x reference)
import jax
import jax.numpy as jnp
from jax.experimental import pallas as pl


def kernel(x, edge_index, edge_attr, batch, params):
    raise NotImplementedError("write your pallas kernel here")


# speedup vs baseline: 1.0728x; 1.0728x over previous
import functools
import numpy as np
import jax
import jax.numpy as jnp
from jax.experimental import pallas as pl
from jax.experimental.pallas import tpu as pltpu

N = 2048
E = 16384
NG = 32
D = 128
AIF = 92
NS = 64
NV = 8
WNUM = 5120
EPS = 1e-5

TE = 256


def _lin(p, x):
    return x @ p[0] + p[1]


def _bn(x, g, b):
    m = jnp.mean(x, 0)
    v = jnp.var(x, 0)
    return (x - m) / jnp.sqrt(v + EPS) * g + b


def _sph(vec):
    u = vec / (jnp.linalg.norm(vec, axis=-1, keepdims=True) + 1e-12)
    x_, y_, z_ = u[:, 0], u[:, 1], u[:, 2]
    sh1 = np.sqrt(3.0) * jnp.stack([y_, z_, x_], -1)
    sh2 = jnp.stack([np.sqrt(15.0) * x_ * y_,
                     np.sqrt(15.0) * y_ * z_,
                     np.sqrt(5.0) / 2.0 * (3.0 * z_ ** 2 - 1.0),
                     np.sqrt(15.0) * x_ * z_,
                     np.sqrt(15.0) / 2.0 * (x_ ** 2 - y_ ** 2)], -1)
    return sh1, sh2


def _equi_fc1_body(ef_ref, hd_ref, sh_ref, a_ref, ab_ref, b_ref, bb_ref,
                   tp_ref):
    h = jax.nn.softplus(
        jnp.dot(ef_ref[:], a_ref[:], preferred_element_type=jnp.float32, precision=jax.lax.Precision.HIGHEST)
        + ab_ref[:])
    w = jnp.dot(h, b_ref[:], preferred_element_type=jnp.float32, precision=jax.lax.Precision.HIGHEST) + bb_ref[:]
    hd = hd_ref[:]
    acc = jnp.zeros((TE, NS), jnp.float32)
    for i in range(NS):
        acc = acc + hd[:, i:i + 1] * w[:, NS * i:NS * (i + 1)]
    a0 = acc * (1.0 / np.sqrt(NS))
    sh1 = sh_ref[:, 0:3]
    sh2 = sh_ref[:, 3:8]
    pieces = [a0]
    for j in range(NV):
        blk = w[:, 4096 + NS * j:4096 + NS * (j + 1)]
        pre = jnp.sum(hd * blk, axis=1, keepdims=True) * (1.0 / np.sqrt(NS))
        pieces.append(pre * sh1)
    for j in range(NV):
        blk = w[:, 4608 + NS * j:4608 + NS * (j + 1)]
        pre = jnp.sum(hd * blk, axis=1, keepdims=True) * (1.0 / np.sqrt(NS))
        pieces.append(pre * sh2)
    tp_ref[:] = jnp.concatenate(pieces, axis=1)


def _equi_fc2_body(ef_ref, od_ref, sh_ref, a_ref, ab_ref, b_ref, bb_ref,
                   tp2_ref):
    h = jax.nn.softplus(
        jnp.dot(ef_ref[:], a_ref[:], preferred_element_type=jnp.float32, precision=jax.lax.Precision.HIGHEST)
        + ab_ref[:])
    w = jnp.dot(h, b_ref[:], preferred_element_type=jnp.float32, precision=jax.lax.Precision.HIGHEST) + bb_ref[:]
    od = od_ref[:]
    h0 = od[:, :NS]
    sh1 = sh_ref[:, 0:3]
    sh2 = sh_ref[:, 3:8]
    acc = jnp.zeros((TE, NS), jnp.float32)
    for i in range(NS):
        acc = acc + h0[:, i:i + 1] * w[:, NS * i:NS * (i + 1)]
    t1 = jnp.zeros((TE, NS), jnp.float32)
    for i in range(NV):
        s1 = jnp.sum(od[:, NS + 3 * i:NS + 3 * (i + 1)] * sh1, axis=1,
                     keepdims=True)
        t1 = t1 + s1 * w[:, 4096 + NS * i:4096 + NS * (i + 1)]
    t2 = jnp.zeros((TE, NS), jnp.float32)
    for i in range(NV):
        s2 = jnp.sum(od[:, NS + 24 + 5 * i:NS + 24 + 5 * (i + 1)] * sh2,
                     axis=1, keepdims=True)
        t2 = t2 + s2 * w[:, 4608 + NS * i:4608 + NS * (i + 1)]
    tp2_ref[:] = (acc + t1 * (1.0 / np.sqrt(3.0)) + t2 * (1.0 / np.sqrt(5.0))
                  ) * (1.0 / np.sqrt(NS + 2 * NV))


def _edge_tile(ncol):
    return pl.BlockSpec((TE, ncol), lambda i: (i, 0))


def _full(shape):
    return pl.BlockSpec(shape, lambda i: tuple(0 for _ in shape))


def _equi_fc1(ef, hd, sh, fc1_a, fc1_ab, fc1_b, fc1_bb):
    return pl.pallas_call(
        _equi_fc1_body,
        grid=(E // TE,),
        in_specs=[_edge_tile(D), _edge_tile(NS), _edge_tile(8),
                  _full((D, D)), _full((1, D)),
                  _full((D, WNUM)), _full((1, WNUM))],
        out_specs=_edge_tile(D),
        out_shape=jax.ShapeDtypeStruct((E, D), jnp.float32),
    )(ef, hd, sh, fc1_a, fc1_ab, fc1_b, fc1_bb)


def _equi_fc2(ef, od, sh, fc2_a, fc2_ab, fc2_b, fc2_bb):
    return pl.pallas_call(
        _equi_fc2_body,
        grid=(E // TE,),
        in_specs=[_edge_tile(D), _edge_tile(D), _edge_tile(8),
                  _full((D, D)), _full((1, D)),
                  _full((D, WNUM)), _full((1, WNUM))],
        out_specs=_edge_tile(NS),
        out_shape=jax.ShapeDtypeStruct((E, NS), jnp.float32),
    )(ef, od, sh, fc2_a, fc2_ab, fc2_b, fc2_bb)


def _perm_fc1_cols():
    sub = np.arange(512).reshape(NS, NV).T.flatten()
    return np.concatenate([np.arange(4096), 4096 + sub, 4608 + sub])


_FC1_PERM = _perm_fc1_cols()


def _equi(eq, nf, src, dst, ef, edge_vec):
    sh1, sh2 = _sph(edge_vec)
    sh = jnp.concatenate([sh1, sh2], axis=1)
    skip = nf
    h = _lin(eq['node_lin'], nf)
    hd = h[dst]
    fc1_b = eq['fc1_b'][0][:, _FC1_PERM]
    fc1_bb = eq['fc1_b'][1][_FC1_PERM][None, :]
    tp = _equi_fc1(ef, hd, sh, eq['fc1_a'][0], eq['fc1_a'][1][None, :],
                   fc1_b, fc1_bb)
    cnt = jnp.clip(jax.ops.segment_sum(jnp.ones((E,), jnp.float32), src,
                                       num_segments=N), 1.0, None)
    o = jax.ops.segment_sum(tp, src, num_segments=N) / cnt[:, None]
    o = o.at[:, :NS].add(h)
    od = o[dst]
    tp2 = _equi_fc2(ef, od, sh, eq['fc2_a'][0], eq['fc2_a'][1][None, :],
                    eq['fc2_b'][0], eq['fc2_b'][1][None, :])
    o2 = jax.ops.segment_sum(tp2, src, num_segments=N) / cnt[:, None]
    o2 = jax.nn.softplus(_lin(eq['node_lin2'],
                              jax.nn.softplus(_bn(o2, eq['bn_g'], eq['bn_b']))))
    return o2 + _lin(eq['skip_lin'], skip)


def _conv(c, x, src, dst, ef):
    q = _lin(c['query'], x)
    k = _lin(c['key'], x)
    v = _lin(c['value'], x)
    ea = _lin(c['edge'], ef)
    kj = _lin(c['keyu2'], jax.nn.silu(
        _lin(c['keyu1'], jnp.concatenate([k[dst], k[src], ea], -1))))
    alpha = q[dst] * kj / np.sqrt(D)
    msg = _lin(c['msg2'], jax.nn.silu(
        _lin(c['msg1'], jnp.concatenate([v[dst], v[src], ea], -1))))
    oe = msg * jax.nn.sigmoid(_bn(alpha, c['bna_g'], c['bna_b']))
    agg = jax.ops.segment_sum(oe, dst, num_segments=x.shape[0])
    out = _lin(c['concate'], agg)
    return jax.nn.softplus(x + _bn(out, c['bn_g'], c['bn_b']))


def _rbf(p, dist):
    centers = jnp.linspace(-4.0, 0.0, D)
    gamma = 1.0 / (4.0 / (D - 1))
    e = jnp.exp(-gamma * (dist[:, None] - centers) ** 2)
    return jax.nn.softplus(_lin(p, e))


def kernel(x, edge_index, edge_attr, batch, params):
    src = edge_index[0]
    dst = edge_index[1]
    nf = _lin(params['atom_emb'], x)
    dist = -0.75 / (jnp.linalg.norm(edge_attr, axis=1) + 1e-12)
    ef = _rbf(params['rbf_lin'], dist)
    nf = _conv(params['convs'][0], nf, src, dst, ef)
    nf = _equi(params['equi'], nf, src, dst, ef, edge_attr)
    nf = _conv(params['convs'][1], nf, src, dst, ef)
    nf = _conv(params['convs'][2], nf, src, dst, ef)
    cnt = jnp.clip(jax.ops.segment_sum(jnp.ones((N,), nf.dtype), batch,
                                       num_segments=NG), 1.0, None)
    feats = jax.ops.segment_sum(nf, batch, num_segments=NG) / cnt[:, None]
    feats = jax.nn.silu(_lin(params['fc'], feats))
    out = _lin(params['fc_out'], feats)
    return jnp.squeeze(out, -1)


# speedup vs baseline: 1.8501x; 1.7245x over previous
import functools
import numpy as np
import jax
import jax.numpy as jnp
from jax import lax
from jax.experimental import pallas as pl
from jax.experimental.pallas import tpu as pltpu
import jax.experimental.pallas.tpu_sc as plsc

N = 2048
E = 16384
NG = 32
D = 128
AIF = 92
NS = 64
NV = 8
WNUM = 5120
EPS = 1e-5

TE = 256


def _lin(p, x):
    return x @ p[0] + p[1]


def _bn(x, g, b):
    m = jnp.mean(x, 0)
    v = jnp.var(x, 0)
    return (x - m) / jnp.sqrt(v + EPS) * g + b


def _sph(vec):
    u = vec / (jnp.linalg.norm(vec, axis=-1, keepdims=True) + 1e-12)
    x_, y_, z_ = u[:, 0], u[:, 1], u[:, 2]
    sh1 = np.sqrt(3.0) * jnp.stack([y_, z_, x_], -1)
    sh2 = jnp.stack([np.sqrt(15.0) * x_ * y_,
                     np.sqrt(15.0) * y_ * z_,
                     np.sqrt(5.0) / 2.0 * (3.0 * z_ ** 2 - 1.0),
                     np.sqrt(15.0) * x_ * z_,
                     np.sqrt(15.0) / 2.0 * (x_ ** 2 - y_ ** 2)], -1)
    return sh1, sh2


_NW = 32
_CH = 128
_PERW = E // _NW
_NCH = _PERW // _CH

_SC_MESH = dict(core_axis_name="c", subcore_axis_name="s")


def _sc_gather(table, idx2):
    Dw = table.shape[1]

    @functools.partial(
        pl.kernel,
        mesh=plsc.VectorSubcoreMesh(**_SC_MESH),
        out_type=jax.ShapeDtypeStruct((E, Dw), jnp.float32),
        scratch_types=[pltpu.VMEM((_NCH, _CH), jnp.int32),
                       pltpu.VMEM((_CH, Dw), jnp.float32),
                       pltpu.SemaphoreType.DMA],
    )
    def k(table_hbm, idx_hbm, out_hbm, idx_v, rows_v, sem):
        wid = lax.axis_index("s") * 2 + lax.axis_index("c")
        pltpu.sync_copy(idx_hbm.at[pl.ds(wid * _NCH, _NCH)], idx_v)
        for j in range(_NCH):
            pltpu.async_copy(table_hbm.at[idx_v.at[j]], rows_v, sem).wait()
            pltpu.sync_copy(
                rows_v, out_hbm.at[pl.ds(wid * _PERW + j * _CH, _CH)])

    return k(table, idx2)


def _sc_scatter_add(vals, idx2, zeros):
    Dw = vals.shape[1]

    @functools.partial(
        pl.kernel,
        mesh=plsc.VectorSubcoreMesh(**_SC_MESH),
        out_type=jax.ShapeDtypeStruct((2 * N, Dw), jnp.float32),
        scratch_types=[pltpu.VMEM((_NCH, _CH), jnp.int32),
                       pltpu.VMEM((_CH, Dw), jnp.float32),
                       pltpu.VMEM_SHARED((N, Dw), jnp.float32)],
    )
    def k(vals_hbm, idx_hbm, zeros_hbm, out_hbm, idx_v, rows_v, acc_sh):
        cid = lax.axis_index("c")
        sid = lax.axis_index("s")
        wid = sid * 2 + cid
        slc = N // 16
        pltpu.sync_copy(zeros_hbm.at[pl.ds(sid * slc, slc)],
                        acc_sh.at[pl.ds(sid * slc, slc)])
        plsc.subcore_barrier()
        pltpu.sync_copy(idx_hbm.at[pl.ds(wid * _NCH, _NCH)], idx_v)
        for j in range(_NCH):
            pltpu.sync_copy(vals_hbm.at[pl.ds(wid * _PERW + j * _CH, _CH)],
                            rows_v)
            pltpu.sync_copy(rows_v, acc_sh.at[idx_v.at[j]], add=True)
        plsc.subcore_barrier()
        pltpu.sync_copy(acc_sh.at[pl.ds(sid * slc, slc)],
                        out_hbm.at[pl.ds(cid * N + sid * slc, slc)])

    return k(vals, idx2, zeros)


def _segment_sum_sc(vals, idx2, zeros):
    p = _sc_scatter_add(vals, idx2, zeros)
    return p[:N] + p[N:]


def _equi_fc1_body(ef_ref, hd_ref, sh_ref, a_ref, ab_ref, b_ref, bb_ref,
                   tp_ref):
    h = jax.nn.softplus(
        jnp.dot(ef_ref[:], a_ref[:], preferred_element_type=jnp.float32, precision=jax.lax.Precision.HIGHEST)
        + ab_ref[:])
    w = jnp.dot(h, b_ref[:], preferred_element_type=jnp.float32, precision=jax.lax.Precision.HIGHEST) + bb_ref[:]
    hd = hd_ref[:, :NS]
    acc = jnp.zeros((TE, NS), jnp.float32)
    for i in range(NS):
        acc = acc + hd[:, i:i + 1] * w[:, NS * i:NS * (i + 1)]
    a0 = acc * (1.0 / np.sqrt(NS))
    sh1 = sh_ref[:, 0:3]
    sh2 = sh_ref[:, 3:8]
    pieces = [a0]
    for j in range(NV):
        blk = w[:, 4096 + NS * j:4096 + NS * (j + 1)]
        pre = jnp.sum(hd * blk, axis=1, keepdims=True) * (1.0 / np.sqrt(NS))
        pieces.append(pre * sh1)
    for j in range(NV):
        blk = w[:, 4608 + NS * j:4608 + NS * (j + 1)]
        pre = jnp.sum(hd * blk, axis=1, keepdims=True) * (1.0 / np.sqrt(NS))
        pieces.append(pre * sh2)
    tp_ref[:] = jnp.concatenate(pieces, axis=1)


def _equi_fc2_body(ef_ref, od_ref, sh_ref, a_ref, ab_ref, b_ref, bb_ref,
                   tp2_ref):
    h = jax.nn.softplus(
        jnp.dot(ef_ref[:], a_ref[:], preferred_element_type=jnp.float32, precision=jax.lax.Precision.HIGHEST)
        + ab_ref[:])
    w = jnp.dot(h, b_ref[:], preferred_element_type=jnp.float32, precision=jax.lax.Precision.HIGHEST) + bb_ref[:]
    od = od_ref[:]
    h0 = od[:, :NS]
    sh1 = sh_ref[:, 0:3]
    sh2 = sh_ref[:, 3:8]
    acc = jnp.zeros((TE, NS), jnp.float32)
    for i in range(NS):
        acc = acc + h0[:, i:i + 1] * w[:, NS * i:NS * (i + 1)]
    t1 = jnp.zeros((TE, NS), jnp.float32)
    for i in range(NV):
        s1 = jnp.sum(od[:, NS + 3 * i:NS + 3 * (i + 1)] * sh1, axis=1,
                     keepdims=True)
        t1 = t1 + s1 * w[:, 4096 + NS * i:4096 + NS * (i + 1)]
    t2 = jnp.zeros((TE, NS), jnp.float32)
    for i in range(NV):
        s2 = jnp.sum(od[:, NS + 24 + 5 * i:NS + 24 + 5 * (i + 1)] * sh2,
                     axis=1, keepdims=True)
        t2 = t2 + s2 * w[:, 4608 + NS * i:4608 + NS * (i + 1)]
    tp2 = (acc + t1 * (1.0 / np.sqrt(3.0)) + t2 * (1.0 / np.sqrt(5.0))
           ) * (1.0 / np.sqrt(NS + 2 * NV))
    tp2_ref[:] = jnp.concatenate([tp2, jnp.zeros((TE, NS), jnp.float32)], 1)


def _edge_tile(ncol):
    return pl.BlockSpec((TE, ncol), lambda i: (i, 0))


def _full(shape):
    return pl.BlockSpec(shape, lambda i: tuple(0 for _ in shape))


def _equi_fc1(ef, hd, sh, fc1_a, fc1_ab, fc1_b, fc1_bb):
    return pl.pallas_call(
        _equi_fc1_body,
        grid=(E // TE,),
        in_specs=[_edge_tile(D), _edge_tile(D), _edge_tile(8),
                  _full((D, D)), _full((1, D)),
                  _full((D, WNUM)), _full((1, WNUM))],
        out_specs=_edge_tile(D),
        out_shape=jax.ShapeDtypeStruct((E, D), jnp.float32),
    )(ef, hd, sh, fc1_a, fc1_ab, fc1_b, fc1_bb)


def _equi_fc2(ef, od, sh, fc2_a, fc2_ab, fc2_b, fc2_bb):
    return pl.pallas_call(
        _equi_fc2_body,
        grid=(E // TE,),
        in_specs=[_edge_tile(D), _edge_tile(D), _edge_tile(8),
                  _full((D, D)), _full((1, D)),
                  _full((D, WNUM)), _full((1, WNUM))],
        out_specs=_edge_tile(D),
        out_shape=jax.ShapeDtypeStruct((E, D), jnp.float32),
    )(ef, od, sh, fc2_a, fc2_ab, fc2_b, fc2_bb)


def _perm_fc1_cols():
    sub = np.arange(512).reshape(NS, NV).T.flatten()
    return np.concatenate([np.arange(4096), 4096 + sub, 4608 + sub])


_FC1_PERM = _perm_fc1_cols()


def _equi(eq, nf, idx2_src, idx2_dst, ef, edge_vec):
    sh1, sh2 = _sph(edge_vec)
    sh = jnp.concatenate([sh1, sh2], axis=1)
    skip = nf
    h = _lin(eq['node_lin'], nf)
    hd = _sc_gather(jnp.concatenate([h, jnp.zeros((N, NS), jnp.float32)], 1),
                    idx2_dst)
    fc1_b = eq['fc1_b'][0][:, _FC1_PERM]
    fc1_bb = eq['fc1_b'][1][_FC1_PERM][None, :]
    tp = _equi_fc1(ef, hd, sh, eq['fc1_a'][0], eq['fc1_a'][1][None, :],
                   fc1_b, fc1_bb)
    cnt = jnp.clip(_segment_sum_sc(jnp.ones((E, D), jnp.float32), idx2_src,
                                   jnp.zeros((N, D), jnp.float32))[:, :1],
                   1.0, None)
    o = _segment_sum_sc(tp, idx2_src, jnp.zeros((N, D), jnp.float32)) / cnt
    o = o.at[:, :NS].add(h)
    od = _sc_gather(o, idx2_dst)
    tp2 = _equi_fc2(ef, od, sh, eq['fc2_a'][0], eq['fc2_a'][1][None, :],
                    eq['fc2_b'][0], eq['fc2_b'][1][None, :])
    o2 = _segment_sum_sc(tp2, idx2_src,
                         jnp.zeros((N, D), jnp.float32))[:, :NS] / cnt
    o2 = jax.nn.softplus(_lin(eq['node_lin2'],
                              jax.nn.softplus(_bn(o2, eq['bn_g'], eq['bn_b']))))
    return o2 + _lin(eq['skip_lin'], skip)


def _conv(c, x, idx2_src, idx2_dst, ef):
    q = _lin(c['query'], x)
    k = _lin(c['key'], x)
    v = _lin(c['value'], x)
    ea = _lin(c['edge'], ef)
    qkv = jnp.concatenate([q, k, v], axis=1)
    kv = qkv[:, D:]
    g_dst = _sc_gather(qkv, idx2_dst)
    g_src = _sc_gather(kv, idx2_src)
    qd, kd, vd = g_dst[:, :D], g_dst[:, D:2 * D], g_dst[:, 2 * D:]
    ks, vs = g_src[:, :D], g_src[:, D:]
    kj = _lin(c['keyu2'], jax.nn.silu(
        _lin(c['keyu1'], jnp.concatenate([kd, ks, ea], -1))))
    alpha = qd * kj / np.sqrt(D)
    msg = _lin(c['msg2'], jax.nn.silu(
        _lin(c['msg1'], jnp.concatenate([vd, vs, ea], -1))))
    oe = msg * jax.nn.sigmoid(_bn(alpha, c['bna_g'], c['bna_b']))
    agg = _segment_sum_sc(oe, idx2_dst, jnp.zeros((N, D), jnp.float32))
    out = _lin(c['concate'], agg)
    return jax.nn.softplus(x + _bn(out, c['bn_g'], c['bn_b']))


def _rbf(p, dist):
    centers = jnp.linspace(-4.0, 0.0, D)
    gamma = 1.0 / (4.0 / (D - 1))
    e = jnp.exp(-gamma * (dist[:, None] - centers) ** 2)
    return jax.nn.softplus(_lin(p, e))


def kernel(x, edge_index, edge_attr, batch, params):
    idx2_src = edge_index[0].astype(jnp.int32).reshape(E // _CH, _CH)
    idx2_dst = edge_index[1].astype(jnp.int32).reshape(E // _CH, _CH)
    nf = _lin(params['atom_emb'], x)
    dist = -0.75 / (jnp.linalg.norm(edge_attr, axis=1) + 1e-12)
    ef = _rbf(params['rbf_lin'], dist)
    nf = _conv(params['convs'][0], nf, idx2_src, idx2_dst, ef)
    nf = _equi(params['equi'], nf, idx2_src, idx2_dst, ef, edge_attr)
    nf = _conv(params['convs'][1], nf, idx2_src, idx2_dst, ef)
    nf = _conv(params['convs'][2], nf, idx2_src, idx2_dst, ef)
    cnt = jnp.clip(jax.ops.segment_sum(jnp.ones((N,), nf.dtype), batch,
                                       num_segments=NG), 1.0, None)
    feats = jax.ops.segment_sum(nf, batch, num_segments=NG) / cnt[:, None]
    feats = jax.nn.silu(_lin(params['fc'], feats))
    out = _lin(params['fc_out'], feats)
    return jnp.squeeze(out, -1)


# speedup vs baseline: 2.6518x; 1.4333x over previous
import functools
import numpy as np
import jax
import jax.numpy as jnp
from jax import lax
from jax.experimental import pallas as pl
from jax.experimental.pallas import tpu as pltpu
import jax.experimental.pallas.tpu_sc as plsc

N = 2048
E = 16384
NG = 32
D = 128
AIF = 92
NS = 64
NV = 8
WNUM = 5120
EPS = 1e-5

TE = 256


def _lin(p, x):
    return x @ p[0] + p[1]


def _bn(x, g, b):
    m = jnp.mean(x, 0)
    v = jnp.var(x, 0)
    return (x - m) / jnp.sqrt(v + EPS) * g + b


def _sph(vec):
    u = vec / (jnp.linalg.norm(vec, axis=-1, keepdims=True) + 1e-12)
    x_, y_, z_ = u[:, 0], u[:, 1], u[:, 2]
    sh1 = np.sqrt(3.0) * jnp.stack([y_, z_, x_], -1)
    sh2 = jnp.stack([np.sqrt(15.0) * x_ * y_,
                     np.sqrt(15.0) * y_ * z_,
                     np.sqrt(5.0) / 2.0 * (3.0 * z_ ** 2 - 1.0),
                     np.sqrt(15.0) * x_ * z_,
                     np.sqrt(15.0) / 2.0 * (x_ ** 2 - y_ ** 2)], -1)
    return sh1, sh2


_NW = 32
_CH = 128
_PERW = E // _NW
_NCH = _PERW // _CH

_SC_MESH = dict(core_axis_name="c", subcore_axis_name="s")


def _sc_gather(table, idx2):
    Dw = table.shape[1]

    @functools.partial(
        pl.kernel,
        mesh=plsc.VectorSubcoreMesh(**_SC_MESH),
        out_type=jax.ShapeDtypeStruct((E, Dw), jnp.float32),
        scratch_types=[pltpu.VMEM((_NCH, _CH), jnp.int32),
                       pltpu.VMEM((_CH, Dw), jnp.float32),
                       pltpu.SemaphoreType.DMA],
    )
    def k(table_hbm, idx_hbm, out_hbm, idx_v, rows_v, sem):
        wid = lax.axis_index("s") * 2 + lax.axis_index("c")
        pltpu.sync_copy(idx_hbm.at[pl.ds(wid * _NCH, _NCH)], idx_v)
        for j in range(_NCH):
            pltpu.async_copy(table_hbm.at[idx_v.at[j]], rows_v, sem).wait()
            pltpu.sync_copy(
                rows_v, out_hbm.at[pl.ds(wid * _PERW + j * _CH, _CH)])

    return k(table, idx2)


def _sc_scatter_add(vals, idx2, zeros):
    Dw = vals.shape[1]

    @functools.partial(
        pl.kernel,
        mesh=plsc.VectorSubcoreMesh(**_SC_MESH),
        out_type=jax.ShapeDtypeStruct((2 * N, Dw), jnp.float32),
        scratch_types=[pltpu.VMEM((_NCH, _CH), jnp.int32),
                       pltpu.VMEM((_CH, Dw), jnp.float32),
                       pltpu.VMEM_SHARED((N, Dw), jnp.float32)],
    )
    def k(vals_hbm, idx_hbm, zeros_hbm, out_hbm, idx_v, rows_v, acc_sh):
        cid = lax.axis_index("c")
        sid = lax.axis_index("s")
        wid = sid * 2 + cid
        slc = N // 16
        pltpu.sync_copy(zeros_hbm.at[pl.ds(sid * slc, slc)],
                        acc_sh.at[pl.ds(sid * slc, slc)])
        plsc.subcore_barrier()
        pltpu.sync_copy(idx_hbm.at[pl.ds(wid * _NCH, _NCH)], idx_v)
        for j in range(_NCH):
            pltpu.sync_copy(vals_hbm.at[pl.ds(wid * _PERW + j * _CH, _CH)],
                            rows_v)
            pltpu.sync_copy(rows_v, acc_sh.at[idx_v.at[j]], add=True)
        plsc.subcore_barrier()
        pltpu.sync_copy(acc_sh.at[pl.ds(sid * slc, slc)],
                        out_hbm.at[pl.ds(cid * N + sid * slc, slc)])

    return k(vals, idx2, zeros)


def _segment_sum_sc(vals, idx2, zeros):
    p = _sc_scatter_add(vals, idx2, zeros)
    return p[:N] + p[N:]


def _split_bf16(x):
    xh = x.astype(jnp.bfloat16)
    xl = (x - xh.astype(jnp.float32)).astype(jnp.bfloat16)
    return xh, xl


def _dot3(x, bh_ref, bl_ref):
    xh, xl = _split_bf16(x)
    bh = bh_ref[:]
    return (jnp.dot(xh, bh, preferred_element_type=jnp.float32)
            + jnp.dot(xh, bl_ref[:], preferred_element_type=jnp.float32)
            + jnp.dot(xl, bh, preferred_element_type=jnp.float32))


def _dot2(x, sel_ref):
    xh, xl = _split_bf16(x)
    sel = sel_ref[:]
    return (jnp.dot(xh, sel, preferred_element_type=jnp.float32)
            + jnp.dot(xl, sel, preferred_element_type=jnp.float32))


def _expand(x, m_ref):
    return _dot2(x, m_ref)


def _foldsum(u, nchunk):
    while nchunk > 1:
        half = u.shape[1] // 2
        u = u[:, :half] + u[:, half:]
        nchunk //= 2
    return u


def _equi_fc1_body(ef_ref, hd_ref, a_ref, ab_ref, bh_ref, bl_ref, bb_ref,
                   sel_ref, shx_ref, m64_ref, mexp_ref, tp_ref):
    HI = jax.lax.Precision.HIGHEST
    h = jax.nn.softplus(
        jnp.dot(ef_ref[:], a_ref[:], preferred_element_type=jnp.float32,
                precision=HI) + ab_ref[:])
    w = _dot3(h, bh_ref, bl_ref) + bb_ref[:]
    hd = hd_ref[:, :NS]
    hd_rep = _expand(hd, m64_ref)
    a0 = _foldsum(w[:, :4096] * hd_rep, NS) * (1.0 / np.sqrt(NS))
    hd_t16 = jnp.tile(hd, (1, 16))
    u = w[:, 4096:5120] * hd_t16
    pre = _dot2(u, sel_ref) * (1.0 / np.sqrt(NS))
    pre_exp = _dot2(pre[:, :16], mexp_ref)
    tp_ref[:, :NS] = a0
    tp_ref[:, NS:] = pre_exp * shx_ref[:]


def _equi_fc2_body(ef_ref, od_ref, a_ref, ab_ref, bh_ref, bl_ref, bb_ref,
                   sel2_ref, shx_ref, m64_ref, m16_ref, tp2_ref):
    HI = jax.lax.Precision.HIGHEST
    h = jax.nn.softplus(
        jnp.dot(ef_ref[:], a_ref[:], preferred_element_type=jnp.float32,
                precision=HI) + ab_ref[:])
    w = _dot3(h, bh_ref, bl_ref) + bb_ref[:]
    od = od_ref[:]
    h0 = od[:, :NS]
    t0 = _foldsum(w[:, :4096] * _expand(h0, m64_ref), NS)
    q = od[:, NS:] * shx_ref[:]
    s16 = _dot2(q, sel2_ref)[:, :16]
    scl = jnp.where(lax.broadcasted_iota(jnp.int32, (TE, 16), 1) < 8,
                    1.0 / np.sqrt(3.0), 1.0 / np.sqrt(5.0))
    s16 = s16 * scl
    t12 = _foldsum(w[:, 4096:] * _expand(s16, m16_ref), 16)
    tp2 = (t0 + t12) * (1.0 / np.sqrt(NS + 2 * NV))
    tp2_ref[:, :NS] = tp2
    tp2_ref[:, NS:] = jnp.zeros((TE, NS), jnp.float32)


def _edge_tile(ncol):
    return pl.BlockSpec((TE, ncol), lambda i: (i, 0))


def _full(shape):
    return pl.BlockSpec(shape, lambda i: tuple(0 for _ in shape))


def _equi_fc1(ef, hd, shx, fc1_a, fc1_ab, fc1_b, fc1_bb):
    bh, bl = _split_bf16(fc1_b)
    return pl.pallas_call(
        _equi_fc1_body,
        grid=(E // TE,),
        in_specs=[_edge_tile(D), _edge_tile(D),
                  _full((D, D)), _full((1, D)),
                  _full((D, WNUM)), _full((D, WNUM)), _full((1, WNUM)),
                  _full((1024, D)), _edge_tile(NS),
                  _full((NS, 4096)), _full((16, NS))],
        out_specs=_edge_tile(D),
        out_shape=jax.ShapeDtypeStruct((E, D), jnp.float32),
    )(ef, hd, fc1_a, fc1_ab, bh, bl, fc1_bb,
      jnp.asarray(_SEL1, jnp.bfloat16), shx,
      jnp.asarray(_M64, jnp.bfloat16), jnp.asarray(_MEXP, jnp.bfloat16))


def _equi_fc2(ef, od, shx, fc2_a, fc2_ab, fc2_b, fc2_bb):
    bh, bl = _split_bf16(fc2_b)
    return pl.pallas_call(
        _equi_fc2_body,
        grid=(E // TE,),
        in_specs=[_edge_tile(D), _edge_tile(D),
                  _full((D, D)), _full((1, D)),
                  _full((D, WNUM)), _full((D, WNUM)), _full((1, WNUM)),
                  _full((NS, D)), _edge_tile(NS),
                  _full((NS, 4096)), _full((16, 1024))],
        out_specs=_edge_tile(D),
        out_shape=jax.ShapeDtypeStruct((E, D), jnp.float32),
    )(ef, od, fc2_a, fc2_ab, bh, bl, fc2_bb,
      jnp.asarray(_SEL2, jnp.bfloat16), shx,
      jnp.asarray(_M64, jnp.bfloat16), jnp.asarray(_M16, jnp.bfloat16))


def _perm_fc1_cols():
    sub = np.arange(512).reshape(NS, NV).T.flatten()
    return np.concatenate([np.arange(4096), 4096 + sub, 4608 + sub])


_FC1_PERM = _perm_fc1_cols()


def _make_sel1():
    m = np.zeros((1024, D), np.float32)
    for j in range(16):
        m[j * NS:(j + 1) * NS, j] = 1.0
    return m


def _make_sel2():
    m = np.zeros((NS, D), np.float32)
    for j in range(NV):
        for mm in range(3):
            m[j * 3 + mm, j] = 1.0
        for mm in range(5):
            m[24 + j * 5 + mm, 8 + j] = 1.0
    return m


def _make_rep(k, r):
    m = np.zeros((k, k * r), np.float32)
    for i in range(k):
        m[i, i * r:(i + 1) * r] = 1.0
    return m


def _make_mexp():
    m = np.zeros((16, NS), np.float32)
    for j in range(NV):
        m[j, j * 3:(j + 1) * 3] = 1.0
        m[8 + j, 24 + j * 5:24 + (j + 1) * 5] = 1.0
    return m


_SEL1 = _make_sel1()
_SEL2 = _make_sel2()
_M64 = _make_rep(NS, NS)
_M16 = _make_rep(16, NS)
_MEXP = _make_mexp()


def _equi(eq, nf, idx2_src, idx2_dst, ef, edge_vec):
    sh1, sh2 = _sph(edge_vec)
    shx = jnp.concatenate([jnp.tile(sh1, (1, NV)), jnp.tile(sh2, (1, NV))],
                          axis=1)
    skip = nf
    h = _lin(eq['node_lin'], nf)
    hd = _sc_gather(jnp.concatenate([h, jnp.zeros((N, NS), jnp.float32)], 1),
                    idx2_dst)
    def _permute_cols(wmat):
        head, w2, w3 = wmat[..., :4096], wmat[..., 4096:4608], wmat[..., 4608:]
        def jmaj(x):
            s = x.shape[:-1]
            return jnp.swapaxes(x.reshape(s + (NS, NV)), -1, -2).reshape(
                s + (512,))
        return jnp.concatenate([head, jmaj(w2), jmaj(w3)], -1)
    fc1_b = _permute_cols(eq['fc1_b'][0])
    fc1_bb = _permute_cols(eq['fc1_b'][1])[None, :]
    tp = _equi_fc1(ef, hd, shx, eq['fc1_a'][0], eq['fc1_a'][1][None, :],
                   fc1_b, fc1_bb)
    cnt = jnp.clip(_segment_sum_sc(jnp.ones((E, D), jnp.float32), idx2_src,
                                   jnp.zeros((N, D), jnp.float32))[:, :1],
                   1.0, None)
    o = _segment_sum_sc(tp, idx2_src, jnp.zeros((N, D), jnp.float32)) / cnt
    o = o.at[:, :NS].add(h)
    od = _sc_gather(o, idx2_dst)
    tp2 = _equi_fc2(ef, od, shx, eq['fc2_a'][0], eq['fc2_a'][1][None, :],
                    eq['fc2_b'][0], eq['fc2_b'][1][None, :])
    o2 = _segment_sum_sc(tp2, idx2_src,
                         jnp.zeros((N, D), jnp.float32))[:, :NS] / cnt
    o2 = jax.nn.softplus(_lin(eq['node_lin2'],
                              jax.nn.softplus(_bn(o2, eq['bn_g'], eq['bn_b']))))
    return o2 + _lin(eq['skip_lin'], skip)


def _conv(c, x, idx2_src, idx2_dst, ef):
    q = _lin(c['query'], x)
    k = _lin(c['key'], x)
    v = _lin(c['value'], x)
    ea = _lin(c['edge'], ef)
    qkv = jnp.concatenate([q, k, v], axis=1)
    kv = qkv[:, D:]
    g_dst = _sc_gather(qkv, idx2_dst)
    g_src = _sc_gather(kv, idx2_src)
    qd, kd, vd = g_dst[:, :D], g_dst[:, D:2 * D], g_dst[:, 2 * D:]
    ks, vs = g_src[:, :D], g_src[:, D:]
    kj = _lin(c['keyu2'], jax.nn.silu(
        _lin(c['keyu1'], jnp.concatenate([kd, ks, ea], -1))))
    alpha = qd * kj / np.sqrt(D)
    msg = _lin(c['msg2'], jax.nn.silu(
        _lin(c['msg1'], jnp.concatenate([vd, vs, ea], -1))))
    oe = msg * jax.nn.sigmoid(_bn(alpha, c['bna_g'], c['bna_b']))
    agg = _segment_sum_sc(oe, idx2_dst, jnp.zeros((N, D), jnp.float32))
    out = _lin(c['concate'], agg)
    return jax.nn.softplus(x + _bn(out, c['bn_g'], c['bn_b']))


def _rbf(p, dist):
    centers = jnp.linspace(-4.0, 0.0, D)
    gamma = 1.0 / (4.0 / (D - 1))
    e = jnp.exp(-gamma * (dist[:, None] - centers) ** 2)
    return jax.nn.softplus(_lin(p, e))


def kernel(x, edge_index, edge_attr, batch, params):
    idx2_src = edge_index[0].astype(jnp.int32).reshape(E // _CH, _CH)
    idx2_dst = edge_index[1].astype(jnp.int32).reshape(E // _CH, _CH)
    nf = _lin(params['atom_emb'], x)
    dist = -0.75 / (jnp.linalg.norm(edge_attr, axis=1) + 1e-12)
    ef = _rbf(params['rbf_lin'], dist)
    nf = _conv(params['convs'][0], nf, idx2_src, idx2_dst, ef)
    nf = _equi(params['equi'], nf, idx2_src, idx2_dst, ef, edge_attr)
    nf = _conv(params['convs'][1], nf, idx2_src, idx2_dst, ef)
    nf = _conv(params['convs'][2], nf, idx2_src, idx2_dst, ef)
    cnt = jnp.clip(jax.ops.segment_sum(jnp.ones((N,), nf.dtype), batch,
                                       num_segments=NG), 1.0, None)
    feats = jax.ops.segment_sum(nf, batch, num_segments=NG) / cnt[:, None]
    feats = jax.nn.silu(_lin(params['fc'], feats))
    out = _lin(params['fc_out'], feats)
    return jnp.squeeze(out, -1)
